# baseline (speedup 1.0000x reference)
import functools

import jax
import jax.numpy as jnp
from jax import lax
from jax.experimental import pallas as pl
from jax.experimental.pallas import tpu as pltpu
from jax.experimental.pallas import tpu_sc as plsc

FDIM = 128
NGRAPH = 16
BLK = 1000
CH = 128
NSC = 2
NTILE = 16
NW = NSC * NTILE
C0_FRAC_NUM = 7
C0_FRAC_DEN = 10


def _make_agg(n_nodes, n_pad, e_pad):
    nch = e_pad // (NW * CH)
    nch0 = ((2 * nch * C0_FRAC_NUM) // (C0_FRAC_DEN * 8)) * 8
    nch1 = 2 * nch - nch0
    rpt = n_pad // NTILE
    mesh = plsc.VectorSubcoreMesh(core_axis_name="c", subcore_axis_name="s")

    scratch_types = [
        pltpu.VMEM((16, CH), jnp.int32),
        pltpu.VMEM((CH, FDIM), jnp.float32),
        pltpu.VMEM_SHARED((n_pad, FDIM), jnp.float32),
        pltpu.SemaphoreType.DMA,
    ]

    @functools.partial(
        pl.kernel,
        out_type=jax.ShapeDtypeStruct((NSC, n_pad, FDIM), jnp.float32),
        mesh=mesh,
        scratch_types=scratch_types,
    )
    def agg(t_hbm, idx_hbm, zero_hbm, out_hbm, idx_v, rows_v, acc_sh, sem):
        c = lax.axis_index("c")
        s = lax.axis_index("s")
        pltpu.sync_copy(zero_hbm, rows_v)
        nfull = rpt // CH
        for i in range(nfull):
            pltpu.sync_copy(rows_v, acc_sh.at[pl.ds(s * rpt + i * CH, CH)])
        rem = rpt - nfull * CH
        if rem:
            pltpu.sync_copy(rows_v.at[pl.ds(0, rem)],
                            acc_sh.at[pl.ds(s * rpt + nfull * CH, rem)])
        plsc.subcore_barrier()

        my_nch = jnp.where(c == 0, nch0, nch1)
        row0 = c * NTILE * nch0 + s * my_nch

        def body(j, carry):
            pltpu.sync_copy(idx_hbm.at[pl.ds(2 * row0 + 16 * j, 16)], idx_v)
            for i in range(8):
                pltpu.async_copy(
                    t_hbm.at[idx_v.at[2 * i]], rows_v, sem).wait()
                pltpu.sync_copy(rows_v, acc_sh.at[idx_v.at[2 * i + 1]],
                                add=True)
            return carry

        lax.fori_loop(0, my_nch // 8, body, 0)

        plsc.subcore_barrier()
        pltpu.sync_copy(acc_sh.at[pl.ds(s * rpt, rpt)],
                        out_hbm.at[c, pl.ds(s * rpt, rpt)])

    return agg


def _enc_body(x_ref, w0_ref, b0_ref, g_ref, bb_ref, h_ref, t_ref):
    h = jnp.dot(x_ref[...], w0_ref[...],
                preferred_element_type=jnp.float32) + b0_ref[...]
    h_ref[...] = h
    mu = jnp.mean(h, axis=-1, keepdims=True)
    var = jnp.mean((h - mu) * (h - mu), axis=-1, keepdims=True)
    t = (h - mu) * lax.rsqrt(var + 1e-5) * g_ref[...] + bb_ref[...]
    t_ref[...] = jnp.maximum(t, 0.0)


def _enc_call(x, w0, b0, g, bb):
    n = x.shape[0]
    nb = n // BLK
    row = pl.BlockSpec((BLK, FDIM), lambda i: (i, 0))
    full = pl.BlockSpec((FDIM, FDIM), lambda i: (0, 0))
    vec = pl.BlockSpec((1, FDIM), lambda i: (0, 0))
    return pl.pallas_call(
        _enc_body,
        grid=(nb,),
        in_specs=[row, full, vec, vec, vec],
        out_specs=(row, row),
        out_shape=(jax.ShapeDtypeStruct((n, FDIM), jnp.float32),
                   jax.ShapeDtypeStruct((n, FDIM), jnp.float32)),
    )(x, w0, b0, g, bb)


def _post_body(h_ref, t_ref, a0_ref, a1_ref, e_ref, w1_ref, s1_ref, f1_ref,
               w2_ref, s2_ref, f2_ref, g_ref, bb_ref, ho_ref, to_ref):
    u = t_ref[...] * e_ref[...] + a0_ref[...] + a1_ref[...]
    z = jnp.dot(u, w1_ref[...], preferred_element_type=jnp.float32)
    z = jnp.maximum(z * s1_ref[...] + f1_ref[...], 0.0)
    z = jnp.dot(z, w2_ref[...], preferred_element_type=jnp.float32)
    z = jnp.maximum(z * s2_ref[...] + f2_ref[...], 0.0)
    h = h_ref[...] + z
    ho_ref[...] = h
    mu = jnp.mean(h, axis=-1, keepdims=True)
    var = jnp.mean((h - mu) * (h - mu), axis=-1, keepdims=True)
    t = (h - mu) * lax.rsqrt(var + 1e-5) * g_ref[...] + bb_ref[...]
    to_ref[...] = jnp.maximum(t, 0.0)


def _post_call(h, t, a0, a1, e, w1, s1, f1, w2, s2, f2, g, bb):
    n = h.shape[0]
    nb = n // BLK
    row = pl.BlockSpec((BLK, FDIM), lambda i: (i, 0))
    full = pl.BlockSpec((FDIM, FDIM), lambda i: (0, 0))
    vec = pl.BlockSpec((1, FDIM), lambda i: (0, 0))
    return pl.pallas_call(
        _post_body,
        grid=(nb,),
        in_specs=[row, row, row, row, vec, full, vec, vec, full, vec, vec,
                  vec, vec],
        out_specs=(row, row),
        out_shape=(jax.ShapeDtypeStruct((n, FDIM), jnp.float32),
                   jax.ShapeDtypeStruct((n, FDIM), jnp.float32)),
    )(h, t, a0, a1, e, w1, s1, f1, w2, s2, f2, g, bb)


def _pool_body(h_ref, b_ref, o_ref, s_acc, c_acc):
    i = pl.program_id(0)
    nb = pl.num_programs(0)

    @pl.when(i == 0)
    def _():
        s_acc[...] = jnp.zeros_like(s_acc)
        c_acc[...] = jnp.zeros_like(c_acc)

    b = jnp.reshape(b_ref[...], (1, BLK))
    oh = (lax.broadcasted_iota(jnp.int32, (NGRAPH, BLK), 0)
          == jnp.broadcast_to(b, (NGRAPH, BLK))).astype(jnp.float32)
    s_acc[...] += jnp.dot(oh, h_ref[...], preferred_element_type=jnp.float32)
    c_acc[...] += jnp.broadcast_to(
        jnp.sum(oh, axis=1, keepdims=True), (NGRAPH, FDIM))

    @pl.when(i == nb - 1)
    def _():
        o_ref[...] = s_acc[...] / jnp.maximum(c_acc[...], 1.0)


def _pool_call(h, batch3d):
    n = h.shape[0]
    nb = n // BLK
    return pl.pallas_call(
        _pool_body,
        grid=(nb,),
        in_specs=[pl.BlockSpec((BLK, FDIM), lambda i: (i, 0)),
                  pl.BlockSpec((1, 1, BLK), lambda i: (i, 0, 0))],
        out_specs=pl.BlockSpec((NGRAPH, FDIM), lambda i: (0, 0)),
        out_shape=jax.ShapeDtypeStruct((NGRAPH, FDIM), jnp.float32),
        scratch_shapes=[pltpu.VMEM((NGRAPH, FDIM), jnp.float32),
                        pltpu.VMEM((NGRAPH, FDIM), jnp.float32)],
    )(h, batch3d)


def kernel(x, edge_index, batch, params):
    n = x.shape[0]
    e = edge_index.shape[1]

    e_pad = -(-e // (NW * CH * 8)) * (NW * CH * 8)
    n_pad = -(-(n + 1) // (NTILE * 8)) * (NTILE * 8)
    src = edge_index[0]
    dst = edge_index[1]
    if e_pad != e:
        pad = e_pad - e
        src = jnp.concatenate([src, jnp.zeros((pad,), jnp.int32)])
        dst = jnp.concatenate(
            [dst, n + (jnp.arange(pad, dtype=jnp.int32) % (n_pad - n))])
    idxs = jnp.reshape(
        jnp.stack([jnp.reshape(src, (e_pad // CH, CH)),
                   jnp.reshape(dst, (e_pad // CH, CH))], axis=1),
        (2 * (e_pad // CH), CH))
    zeros_hbm = jnp.zeros((CH, FDIM), jnp.float32)
    agg_fn = _make_agg(n, n_pad, e_pad)

    def vrow(v):
        return jnp.reshape(v, (1, FDIM))

    p0 = params['layers'][0]
    h, t = _enc_call(x, params['W0'], vrow(params['b0']),
                     vrow(p0['ln_g']), vrow(p0['ln_b']))

    bn_scale = 1.0 / jnp.sqrt(jnp.float32(1.0 + 1e-5))
    nlayers = len(params['layers'])
    for l, p in enumerate(params['layers']):
        parts = agg_fn(t, idxs, zeros_hbm)
        a0 = parts[0]
        a1 = parts[1]
        e_b = jnp.broadcast_to(jnp.reshape(1.0 + p['eps'], (1, 1)), (1, FDIM))
        s1 = vrow(p['bn1_g'] * bn_scale)
        f1 = vrow(p['b1'] * p['bn1_g'] * bn_scale + p['bn1_b'])
        s2 = vrow(p['bn2_g'] * bn_scale)
        f2 = vrow(p['b2'] * p['bn2_g'] * bn_scale + p['bn2_b'])
        pn = params['layers'][l + 1] if l + 1 < nlayers else p
        h, t = _post_call(h, t, a0, a1, e_b, p['W1'], s1, f1,
                          p['W2'], s2, f2, vrow(pn['ln_g']), vrow(pn['ln_b']))

    return _pool_call(h, jnp.reshape(batch, (n // BLK, 1, BLK)))


# speedup vs baseline: 1.2784x; 1.2784x over previous
import functools

import jax
import jax.numpy as jnp
from jax import lax
from jax.experimental import pallas as pl
from jax.experimental.pallas import tpu as pltpu
from jax.experimental.pallas import tpu_sc as plsc

FDIM = 128
NGRAPH = 16
BLK = 1000
CH = 128
NSC = 2
NTILE = 16
NW = NSC * NTILE
C0_FRAC_NUM = 7
C0_FRAC_DEN = 10


def _make_agg(n_nodes, n_pad, e_pad):
    nch = e_pad // (NW * CH)
    nch0 = (2 * nch * C0_FRAC_NUM) // C0_FRAC_DEN
    nch1 = 2 * nch - nch0
    rpt = n_pad // NTILE
    mesh = plsc.VectorSubcoreMesh(core_axis_name="c", subcore_axis_name="s")

    scratch_types = [
        pltpu.VMEM((2, CH), jnp.int32),
        pltpu.VMEM((CH, FDIM), jnp.float32),
        pltpu.VMEM_SHARED((n_pad, FDIM), jnp.float32),
        pltpu.SemaphoreType.DMA,
    ]

    @functools.partial(
        pl.kernel,
        out_type=jax.ShapeDtypeStruct((NSC, n_pad, FDIM), jnp.float32),
        mesh=mesh,
        scratch_types=scratch_types,
    )
    def agg(t_hbm, idx_hbm, zero_hbm, out_hbm, idx_v, rows_v, acc_sh, sem):
        c = lax.axis_index("c")
        s = lax.axis_index("s")
        pltpu.sync_copy(zero_hbm, rows_v)
        nfull = rpt // CH
        for i in range(nfull):
            pltpu.sync_copy(rows_v, acc_sh.at[pl.ds(s * rpt + i * CH, CH)])
        rem = rpt - nfull * CH
        if rem:
            pltpu.sync_copy(rows_v.at[pl.ds(0, rem)],
                            acc_sh.at[pl.ds(s * rpt + nfull * CH, rem)])
        plsc.subcore_barrier()

        my_nch = jnp.where(c == 0, nch0, nch1)
        row0 = c * NTILE * nch0 + s * my_nch

        def body(k, carry):
            pltpu.sync_copy(idx_hbm.at[row0 + k], idx_v)
            pltpu.async_copy(t_hbm.at[idx_v.at[0]], rows_v, sem).wait()
            pltpu.sync_copy(rows_v, acc_sh.at[idx_v.at[1]], add=True)
            return carry

        lax.fori_loop(0, my_nch, body, 0)

        plsc.subcore_barrier()
        pltpu.sync_copy(acc_sh.at[pl.ds(s * rpt, rpt)],
                        out_hbm.at[c, pl.ds(s * rpt, rpt)])

    return agg


def _enc_body(x_ref, w0_ref, b0_ref, g_ref, bb_ref, h_ref, t_ref):
    h = jnp.dot(x_ref[...], w0_ref[...],
                preferred_element_type=jnp.float32) + b0_ref[...]
    h_ref[...] = h
    mu = jnp.mean(h, axis=-1, keepdims=True)
    var = jnp.mean((h - mu) * (h - mu), axis=-1, keepdims=True)
    t = (h - mu) * lax.rsqrt(var + 1e-5) * g_ref[...] + bb_ref[...]
    t_ref[...] = jnp.maximum(t, 0.0)


def _enc_call(x, w0, b0, g, bb):
    n = x.shape[0]
    nb = n // BLK
    row = pl.BlockSpec((BLK, FDIM), lambda i: (i, 0))
    full = pl.BlockSpec((FDIM, FDIM), lambda i: (0, 0))
    vec = pl.BlockSpec((1, FDIM), lambda i: (0, 0))
    return pl.pallas_call(
        _enc_body,
        grid=(nb,),
        in_specs=[row, full, vec, vec, vec],
        out_specs=(row, row),
        out_shape=(jax.ShapeDtypeStruct((n, FDIM), jnp.float32),
                   jax.ShapeDtypeStruct((n, FDIM), jnp.float32)),
    )(x, w0, b0, g, bb)


def _post_body(h_ref, t_ref, a0_ref, a1_ref, e_ref, w1_ref, s1_ref, f1_ref,
               w2_ref, s2_ref, f2_ref, g_ref, bb_ref, ho_ref, to_ref):
    u = t_ref[...] * e_ref[...] + a0_ref[...] + a1_ref[...]
    z = jnp.dot(u, w1_ref[...], preferred_element_type=jnp.float32)
    z = jnp.maximum(z * s1_ref[...] + f1_ref[...], 0.0)
    z = jnp.dot(z, w2_ref[...], preferred_element_type=jnp.float32)
    z = jnp.maximum(z * s2_ref[...] + f2_ref[...], 0.0)
    h = h_ref[...] + z
    ho_ref[...] = h
    mu = jnp.mean(h, axis=-1, keepdims=True)
    var = jnp.mean((h - mu) * (h - mu), axis=-1, keepdims=True)
    t = (h - mu) * lax.rsqrt(var + 1e-5) * g_ref[...] + bb_ref[...]
    to_ref[...] = jnp.maximum(t, 0.0)


def _post_call(h, t, a0, a1, e, w1, s1, f1, w2, s2, f2, g, bb):
    n = h.shape[0]
    nb = n // BLK
    row = pl.BlockSpec((BLK, FDIM), lambda i: (i, 0))
    full = pl.BlockSpec((FDIM, FDIM), lambda i: (0, 0))
    vec = pl.BlockSpec((1, FDIM), lambda i: (0, 0))
    return pl.pallas_call(
        _post_body,
        grid=(nb,),
        in_specs=[row, row, row, row, vec, full, vec, vec, full, vec, vec,
                  vec, vec],
        out_specs=(row, row),
        out_shape=(jax.ShapeDtypeStruct((n, FDIM), jnp.float32),
                   jax.ShapeDtypeStruct((n, FDIM), jnp.float32)),
    )(h, t, a0, a1, e, w1, s1, f1, w2, s2, f2, g, bb)


def _pool_body(h_ref, b_ref, o_ref, s_acc, c_acc):
    i = pl.program_id(0)
    nb = pl.num_programs(0)

    @pl.when(i == 0)
    def _():
        s_acc[...] = jnp.zeros_like(s_acc)
        c_acc[...] = jnp.zeros_like(c_acc)

    b = jnp.reshape(b_ref[...], (1, BLK))
    oh = (lax.broadcasted_iota(jnp.int32, (NGRAPH, BLK), 0)
          == jnp.broadcast_to(b, (NGRAPH, BLK))).astype(jnp.float32)
    s_acc[...] += jnp.dot(oh, h_ref[...], preferred_element_type=jnp.float32)
    c_acc[...] += jnp.broadcast_to(
        jnp.sum(oh, axis=1, keepdims=True), (NGRAPH, FDIM))

    @pl.when(i == nb - 1)
    def _():
        o_ref[...] = s_acc[...] / jnp.maximum(c_acc[...], 1.0)


def _pool_call(h, batch3d):
    n = h.shape[0]
    nb = n // BLK
    return pl.pallas_call(
        _pool_body,
        grid=(nb,),
        in_specs=[pl.BlockSpec((BLK, FDIM), lambda i: (i, 0)),
                  pl.BlockSpec((1, 1, BLK), lambda i: (i, 0, 0))],
        out_specs=pl.BlockSpec((NGRAPH, FDIM), lambda i: (0, 0)),
        out_shape=jax.ShapeDtypeStruct((NGRAPH, FDIM), jnp.float32),
        scratch_shapes=[pltpu.VMEM((NGRAPH, FDIM), jnp.float32),
                        pltpu.VMEM((NGRAPH, FDIM), jnp.float32)],
    )(h, batch3d)


def kernel(x, edge_index, batch, params):
    n = x.shape[0]
    e = edge_index.shape[1]

    e_pad = -(-e // (NW * CH)) * (NW * CH)
    n_pad = -(-(n + 1) // (NTILE * 8)) * (NTILE * 8)
    src = edge_index[0]
    dst = edge_index[1]
    if e_pad != e:
        pad = e_pad - e
        src = jnp.concatenate([src, jnp.zeros((pad,), jnp.int32)])
        dst = jnp.concatenate(
            [dst, n + (jnp.arange(pad, dtype=jnp.int32) % (n_pad - n))])
    idxs = jnp.stack([jnp.reshape(src, (e_pad // CH, CH)),
                      jnp.reshape(dst, (e_pad // CH, CH))], axis=1)
    zeros_hbm = jnp.zeros((CH, FDIM), jnp.float32)
    agg_fn = _make_agg(n, n_pad, e_pad)

    def vrow(v):
        return jnp.reshape(v, (1, FDIM))

    p0 = params['layers'][0]
    h, t = _enc_call(x, params['W0'], vrow(params['b0']),
                     vrow(p0['ln_g']), vrow(p0['ln_b']))

    bn_scale = 1.0 / jnp.sqrt(jnp.float32(1.0 + 1e-5))
    nlayers = len(params['layers'])
    for l, p in enumerate(params['layers']):
        parts = agg_fn(t, idxs, zeros_hbm)
        a0 = parts[0]
        a1 = parts[1]
        e_b = jnp.broadcast_to(jnp.reshape(1.0 + p['eps'], (1, 1)), (1, FDIM))
        s1 = vrow(p['bn1_g'] * bn_scale)
        f1 = vrow(p['b1'] * p['bn1_g'] * bn_scale + p['bn1_b'])
        s2 = vrow(p['bn2_g'] * bn_scale)
        f2 = vrow(p['b2'] * p['bn2_g'] * bn_scale + p['bn2_b'])
        pn = params['layers'][l + 1] if l + 1 < nlayers else p
        h, t = _post_call(h, t, a0, a1, e_b, p['W1'], s1, f1,
                          p['W2'], s2, f2, vrow(pn['ln_g']), vrow(pn['ln_b']))

    return _pool_call(h, jnp.reshape(batch, (n // BLK, 1, BLK)))
